# baseline (speedup 1.0000x reference)
import functools

import jax
import jax.numpy as jnp
from jax.experimental import pallas as pl
from jax.experimental.pallas import tpu as pltpu

_LANE = 128
_SUBLANE = 8
_VMEM_LIMIT_BYTES = 64 * 1024 * 1024


def _round_up(a, m):
    return ((a + m - 1) // m) * m


def _head_kernel(x_ref, s_ref, wf_ref, bf_ref, wc_ref, bc_ref, o_ref, *,
                 groups, inv_hw):
    part = jax.lax.dot_general(
        x_ref[...], s_ref[...], (((1,), (0,)), ((), ())),
        preferred_element_type=jnp.float32)
    part = part * inv_hw
    tn = part.shape[0] // groups
    p3 = part.reshape(tn, groups, _LANE)
    acc = jnp.zeros((tn, wf_ref.shape[1]), jnp.float32)
    for gi in range(groups):
        acc = acc + jnp.dot(p3[:, gi, :],
                            wf_ref[gi * _LANE:(gi + 1) * _LANE, :],
                            preferred_element_type=jnp.float32)
    feat = jnp.maximum(acc + bf_ref[...], 0.0)
    o_ref[...] = jnp.dot(feat, wc_ref[...],
                         preferred_element_type=jnp.float32) + bc_ref[...]


def kernel(x, w_feat, b_feat, w_cls, b_cls):
    n, c, h, w = x.shape
    hw = h * w
    f = w_feat.shape[1]
    k = w_cls.shape[1]
    assert c % _LANE == 0
    groups = c // _LANE
    gsz = _LANE * hw

    fp = _round_up(f, _LANE)
    kp = _round_up(k, _LANE)

    tn = min(32, _round_up(n, _SUBLANE))
    n_pad = _round_up(n, tn)

    x2 = jnp.reshape(x, (n * groups, gsz))
    if n_pad > n:
        x2 = jnp.pad(x2, ((0, (n_pad - n) * groups), (0, 0)))

    q = jax.lax.broadcasted_iota(jnp.int32, (gsz, _LANE), 0)
    j = jax.lax.broadcasted_iota(jnp.int32, (gsz, _LANE), 1)
    s = (q // hw == j).astype(jnp.float32)

    wf = jnp.pad(w_feat, ((0, 0), (0, fp - f)))
    bf = jnp.pad(b_feat, ((0, 0), (0, fp - f)))
    wc = jnp.pad(w_cls, ((0, fp - f), (0, kp - k)))
    bc = jnp.pad(b_cls, ((0, 0), (0, kp - k)))

    tb = tn * groups
    cost = pl.CostEstimate(
        flops=2 * n_pad * groups * gsz * _LANE
        + 2 * n_pad * c * fp + 2 * n_pad * fp * kp,
        transcendentals=0,
        bytes_accessed=4 * (x2.size + s.size + wf.size + wc.size
                            + n_pad * kp),
    )

    out = pl.pallas_call(
        functools.partial(_head_kernel, groups=groups, inv_hw=1.0 / float(hw)),
        out_shape=jax.ShapeDtypeStruct((n_pad, kp), jnp.float32),
        grid=(n_pad // tn,),
        in_specs=[
            pl.BlockSpec((tb, gsz), lambda i: (i, 0)),
            pl.BlockSpec((gsz, _LANE), lambda i: (0, 0)),
            pl.BlockSpec((c, fp), lambda i: (0, 0)),
            pl.BlockSpec((1, fp), lambda i: (0, 0)),
            pl.BlockSpec((fp, kp), lambda i: (0, 0)),
            pl.BlockSpec((1, kp), lambda i: (0, 0)),
        ],
        out_specs=pl.BlockSpec((tn, kp), lambda i: (i, 0)),
        compiler_params=pltpu.CompilerParams(
            dimension_semantics=("parallel",),
            vmem_limit_bytes=_VMEM_LIMIT_BYTES,
        ),
        cost_estimate=cost,
    )(x2, s, wf, bf, wc, bc)
    return {"logits": out[:n, :k]}


# speedup vs baseline: 7.8383x; 7.8383x over previous
import functools

import jax
import jax.numpy as jnp
from jax.experimental import pallas as pl
from jax.experimental.pallas import tpu as pltpu

_LANE = 128
_SUBLANE = 8
_VMEM_LIMIT_BYTES = 64 * 1024 * 1024


def _round_up(a, m):
    return ((a + m - 1) // m) * m


def _head_kernel(x_ref, wf_ref, bf_ref, wc_ref, bc_ref, o_ref, *, inv_hw):
    pooled = jnp.sum(x_ref[...], axis=1) * inv_hw
    feat = jnp.dot(pooled, wf_ref[...],
                   preferred_element_type=jnp.float32)
    feat = jnp.maximum(feat + bf_ref[...], 0.0)
    o_ref[...] = jnp.dot(feat, wc_ref[...],
                         preferred_element_type=jnp.float32) + bc_ref[...]


def kernel(x, w_feat, b_feat, w_cls, b_cls):
    n, c, h, w = x.shape
    hw = h * w
    f = w_feat.shape[1]
    k = w_cls.shape[1]

    fp = _round_up(f, _LANE)
    kp = _round_up(k, _LANE)

    tn = min(32, _round_up(n, _SUBLANE))
    n_pad = _round_up(n, tn)

    xt = jnp.transpose(x, (0, 2, 3, 1)).reshape(n, hw, c)
    if n_pad > n:
        xt = jnp.pad(xt, ((0, n_pad - n), (0, 0), (0, 0)))

    wf = jnp.pad(w_feat, ((0, 0), (0, fp - f)))
    bf = jnp.pad(b_feat, ((0, 0), (0, fp - f)))
    wc = jnp.pad(w_cls, ((0, fp - f), (0, kp - k)))
    bc = jnp.pad(b_cls, ((0, 0), (0, kp - k)))

    cost = pl.CostEstimate(
        flops=2 * n_pad * c * fp + 2 * n_pad * fp * kp,
        transcendentals=0,
        bytes_accessed=4 * (xt.size + wf.size + wc.size + n_pad * kp),
    )

    out = pl.pallas_call(
        functools.partial(_head_kernel, inv_hw=1.0 / float(hw)),
        out_shape=jax.ShapeDtypeStruct((n_pad, kp), jnp.float32),
        grid=(n_pad // tn,),
        in_specs=[
            pl.BlockSpec((tn, hw, c), lambda i: (i, 0, 0)),
            pl.BlockSpec((c, fp), lambda i: (0, 0)),
            pl.BlockSpec((1, fp), lambda i: (0, 0)),
            pl.BlockSpec((fp, kp), lambda i: (0, 0)),
            pl.BlockSpec((1, kp), lambda i: (0, 0)),
        ],
        out_specs=pl.BlockSpec((tn, kp), lambda i: (i, 0)),
        compiler_params=pltpu.CompilerParams(
            dimension_semantics=("parallel",),
            vmem_limit_bytes=_VMEM_LIMIT_BYTES,
        ),
        cost_estimate=cost,
    )(xt, wf, bf, wc, bc)
    return {"logits": out[:n, :k]}
